# baseline (speedup 1.0000x reference)
import functools

import jax
import jax.numpy as jnp
from jax.experimental import pallas as pl

B = 256
NSLOTS = 128
M = NSLOTS - 1
D_EMB = 128
D_QUERY = 512
TB = 8

_F32 = jnp.float32
_BF16 = jnp.bfloat16


def _ln(h, g, b, dot_unused=None):
    m = jnp.mean(h, axis=-1, keepdims=True)
    v = jnp.mean((h - m) ** 2, axis=-1, keepdims=True)
    return (h - m) * jax.lax.rsqrt(v + 1e-5) * g + b


def _hi_lo(a):
    hi = a.astype(_BF16)
    lo = (a - hi.astype(_F32)).astype(_BF16)
    return hi, lo


def _fused_kernel(
    q_ref, x_ref, ip_ref, il_ref, ir_ref,
    q1w, q1b, q1g, q1be, q2w, q2b, q2g, q2be, q3w, q3b,
    w1x_hi, w1x_lo, w1q, b1, w2_hi, w2_lo, b2, w3_hi, w3_lo, b3,
    f1w, f1b, f1g, f1be, f2w, f2b, f2g, f2be, f3w, f3b, f3g, f3be, f4w, f4b,
    out_ref,
):
    dot = functools.partial(jnp.dot, preferred_element_type=_F32)
    dotH = functools.partial(
        jnp.dot, preferred_element_type=_F32, precision=jax.lax.Precision.HIGHEST
    )

    h = jax.nn.relu(_ln(dotH(q_ref[...], q1w[...]) + q1b[...], q1g[...], q1be[...]))
    h = jax.nn.relu(_ln(dotH(h, q2w[...]) + q2b[...], q2g[...], q2be[...]))
    qf = dotH(h, q3w[...]) + q3b[...]
    qproj = dotH(qf, w1q[...])

    iota2 = jax.lax.broadcasted_iota(jnp.int32, (NSLOTS, 2 * NSLOTS), 1) & (NSLOTS - 1)
    rmask = jax.lax.broadcasted_iota(jnp.int32, (NSLOTS, 1), 0) > 0
    ohs = []
    for t in range(TB):
        oh3 = []
        for ref in (ip_ref, il_ref, ir_ref):
            col = ref[0, :, t : t + 1]
            oh3.append(((col == iota2) & rmask).astype(_BF16))
        ohs.append(oh3)
    rowmask = rmask.astype(_F32)

    xa = x_ref[...]
    xf = xa.reshape(TB * NSLOTS, D_EMB)
    xh, xl = _hi_lo(xf)
    s = dot(xh, w1x_hi[...]) + dot(xh, w1x_lo[...]) + dot(xl, w1x_hi[...])
    s = s.reshape(TB, NSLOTS, 3 * 512) + qproj[:, None, :]

    def gather_layer(s3, co, bias):
        hi, lo = _hi_lo(s3)
        z = jnp.concatenate([hi, lo], axis=1)
        outs = []
        for t in range(TB):
            zt = z[t]
            r = (
                dot(ohs[t][0], zt[:, :co])
                + dot(ohs[t][1], zt[:, co : 2 * co])
                + dot(ohs[t][2], zt[:, 2 * co :])
            )
            outs.append(r[None])
        return jnp.concatenate(outs, axis=0) + (rowmask * bias[...])[None]

    def tln(t):
        n = NSLOTS * t.shape[2]
        m = jnp.mean(t, axis=(1, 2), keepdims=True)
        d = t - m
        v = jnp.sum(d * d, axis=(1, 2), keepdims=True) / (n - 1)
        return d / (jnp.sqrt(v) + 0.001)

    t1 = tln(gather_layer(s, 512, b1))

    a_hi, a_lo = _hi_lo(t1.reshape(TB * NSLOTS, 512))
    s2 = dot(a_hi, w2_hi[...]) + dot(a_hi, w2_lo[...]) + dot(a_lo, w2_hi[...])
    t2 = tln(gather_layer(s2.reshape(TB, NSLOTS, 3 * 256), 256, b2))

    a_hi, a_lo = _hi_lo(t2.reshape(TB * NSLOTS, 256))
    s3 = dot(a_hi, w3_hi[...]) + dot(a_hi, w3_lo[...]) + dot(a_lo, w3_hi[...])
    t3 = gather_layer(s3.reshape(TB, NSLOTS, 3 * 128), 128, b3)

    pooled = jnp.max(t3, axis=1)
    h = jax.nn.relu(_ln(dotH(pooled, f1w[...]) + f1b[...], f1g[...], f1be[...]))
    h = jax.nn.relu(_ln(dotH(h, f2w[...]) + f2b[...], f2g[...], f2be[...]))
    h = jax.nn.relu(_ln(dotH(h, f3w[...]) + f3b[...], f3g[...], f3be[...]))
    out_ref[...] = dotH(h, f4w[...]) + f4b[...]


def kernel(q, x, indices, lens, params):
    p = params
    idx = indices[:, :, 0]
    zrow = jnp.zeros((1, B), jnp.int32)
    grp = lambda a: a.reshape(NSLOTS, B // TB, TB).transpose(1, 0, 2)
    ip = grp(jnp.concatenate([zrow, idx[0::3]], axis=0))
    il = grp(jnp.concatenate([zrow, idx[1::3]], axis=0))
    ir = grp(jnp.concatenate([zrow, idx[2::3]], axis=0))

    xt = jnp.transpose(x, (1, 0, 2))

    c = D_EMB + 32
    w1 = p["c1w"]
    w1p, w1l, w1r = w1[:c], w1[c : 2 * c], w1[2 * c :]
    w1x = jnp.concatenate([w1p[:D_EMB], w1l[:D_EMB], w1r[:D_EMB]], axis=1)
    w1q = jnp.concatenate([w1p[D_EMB:], w1l[D_EMB:], w1r[D_EMB:]], axis=1)
    w2 = p["c2w"]
    w2c = jnp.concatenate([w2[:512], w2[512:1024], w2[1024:]], axis=1)
    w3 = p["c3w"]
    w3c = jnp.concatenate([w3[:256], w3[256:512], w3[512:]], axis=1)

    hi_lo = lambda a: (a.astype(_BF16), (a - a.astype(_BF16).astype(_F32)).astype(_BF16))
    w1x_hi, w1x_lo = hi_lo(w1x)
    w2_hi, w2_lo = hi_lo(w2c)
    w3_hi, w3_lo = hi_lo(w3c)

    row2d = lambda a: a.reshape(1, -1)
    weights = [
        p["q1w"], row2d(p["q1b"]), row2d(p["q1g"]), row2d(p["q1be"]),
        p["q2w"], row2d(p["q2b"]), row2d(p["q2g"]), row2d(p["q2be"]),
        p["q3w"], row2d(p["q3b"]),
        w1x_hi, w1x_lo, w1q, row2d(p["c1b"]),
        w2_hi, w2_lo, row2d(p["c2b"]),
        w3_hi, w3_lo, row2d(p["c3b"]),
        p["f1w"], row2d(p["f1b"]), row2d(p["f1g"]), row2d(p["f1be"]),
        p["f2w"], row2d(p["f2b"]), row2d(p["f2g"]), row2d(p["f2be"]),
        p["f3w"], row2d(p["f3b"]), row2d(p["f3g"]), row2d(p["f3be"]),
        p["f4w"], row2d(p["f4b"]),
    ]

    full = lambda shape: pl.BlockSpec(shape, lambda i: (0,) * len(shape))
    in_specs = [
        pl.BlockSpec((TB, D_QUERY), lambda i: (i, 0)),
        pl.BlockSpec((TB, NSLOTS, D_EMB), lambda i: (i, 0, 0)),
        pl.BlockSpec((1, NSLOTS, TB), lambda i: (i, 0, 0)),
        pl.BlockSpec((1, NSLOTS, TB), lambda i: (i, 0, 0)),
        pl.BlockSpec((1, NSLOTS, TB), lambda i: (i, 0, 0)),
    ]
    in_specs += [full(w.shape) for w in weights]

    out = pl.pallas_call(
        _fused_kernel,
        grid=(B // TB,),
        in_specs=in_specs,
        out_specs=pl.BlockSpec((TB, 1), lambda i: (i, 0)),
        out_shape=jax.ShapeDtypeStruct((B, 1), jnp.float32),
    )(q, xt, ip, il, ir, *weights)
    return out * lens[0].astype(out.dtype)


# speedup vs baseline: 1.2918x; 1.2918x over previous
import functools

import jax
import jax.numpy as jnp
from jax.experimental import pallas as pl

B = 256
NSLOTS = 128
M = NSLOTS - 1
D_EMB = 128
D_QUERY = 512
TB = 8

_F32 = jnp.float32
_BF16 = jnp.bfloat16


def _split3(a):
    hi = a.astype(_BF16)
    r1 = a - hi.astype(_F32)
    mid = r1.astype(_BF16)
    lo = (r1 - mid.astype(_F32)).astype(_BF16)
    return hi, mid, lo


def _fused_kernel(
    q_ref, x_ref, ip_ref, il_ref, ir_ref,
    q1w, q1b, q1g, q1be, q2w, q2b, q2g, q2be, q3w, q3b,
    w1, b1, w2, b2, w3, b3,
    f1w, f1b, f1g, f1be, f2w, f2b, f2g, f2be, f3w, f3b, f3g, f3be, f4w, f4b,
    out_ref,
):
    dot = functools.partial(jnp.dot, preferred_element_type=_F32)

    def ln(h, g, b):
        m = jnp.mean(h, axis=-1, keepdims=True)
        v = jnp.mean((h - m) ** 2, axis=-1, keepdims=True)
        return (h - m) / jnp.sqrt(v + 1e-5) * g + b

    h = jax.nn.relu(ln(dot(q_ref[...], q1w[...]) + q1b[...], q1g[...], q1be[...]))
    h = jax.nn.relu(ln(dot(h, q2w[...]) + q2b[...], q2g[...], q2be[...]))
    qf = dot(h, q3w[...]) + q3b[...]

    iota2 = jax.lax.broadcasted_iota(jnp.int32, (NSLOTS, 3 * NSLOTS), 1) & (NSLOTS - 1)
    rmask = jax.lax.broadcasted_iota(jnp.int32, (NSLOTS, 1), 0) > 0
    ohs = []
    for t in range(TB):
        oh3 = []
        for ref in (ip_ref, il_ref, ir_ref):
            col = ref[0, :, t : t + 1]
            oh3.append(((col == iota2) & rmask).astype(_BF16))
        ohs.append(oh3)
    rowmask = rmask.astype(_F32)

    xa = x_ref[...]
    trees = jnp.concatenate(
        [xa, jnp.broadcast_to(qf[:, None, :], (TB, NSLOTS, 32))], axis=2
    )

    def conv(tin, w, bias):
        cin = tin.shape[2]
        z = jnp.concatenate(_split3(tin), axis=1)
        rows = []
        for t in range(TB):
            zt = z[t]
            rows.append(
                jnp.concatenate(
                    [dot(ohs[t][0], zt), dot(ohs[t][1], zt), dot(ohs[t][2], zt)],
                    axis=1,
                )[None]
            )
        g = jnp.concatenate(rows, axis=0).reshape(TB * NSLOTS, 3 * cin)
        s = dot(g, w[...]).reshape(TB, NSLOTS, -1)
        return s + (rowmask * bias[...])[None]

    def tln(t):
        n = NSLOTS * t.shape[2]
        m = jnp.mean(t, axis=(1, 2), keepdims=True)
        d = t - m
        s = jnp.sqrt(jnp.sum(d * d, axis=(1, 2), keepdims=True) / (n - 1))
        return d / (s + 0.001)

    t1 = tln(conv(trees, w1, b1))
    t2 = tln(conv(t1, w2, b2))
    t3 = conv(t2, w3, b3)

    pooled = jnp.max(t3, axis=1)
    h = jax.nn.relu(ln(dot(pooled, f1w[...]) + f1b[...], f1g[...], f1be[...]))
    h = jax.nn.relu(ln(dot(h, f2w[...]) + f2b[...], f2g[...], f2be[...]))
    h = jax.nn.relu(ln(dot(h, f3w[...]) + f3b[...], f3g[...], f3be[...]))
    out_ref[...] = dot(h, f4w[...]) + f4b[...]


def kernel(q, x, indices, lens, params):
    p = params
    idx = indices[:, :, 0]
    zrow = jnp.zeros((1, B), jnp.int32)
    grp = lambda a: a.reshape(NSLOTS, B // TB, TB).transpose(1, 0, 2)
    ip = grp(jnp.concatenate([zrow, idx[0::3]], axis=0))
    il = grp(jnp.concatenate([zrow, idx[1::3]], axis=0))
    ir = grp(jnp.concatenate([zrow, idx[2::3]], axis=0))

    xt = jnp.transpose(x, (1, 0, 2))

    row2d = lambda a: a.reshape(1, -1)
    weights = [
        p["q1w"], row2d(p["q1b"]), row2d(p["q1g"]), row2d(p["q1be"]),
        p["q2w"], row2d(p["q2b"]), row2d(p["q2g"]), row2d(p["q2be"]),
        p["q3w"], row2d(p["q3b"]),
        p["c1w"], row2d(p["c1b"]),
        p["c2w"], row2d(p["c2b"]),
        p["c3w"], row2d(p["c3b"]),
        p["f1w"], row2d(p["f1b"]), row2d(p["f1g"]), row2d(p["f1be"]),
        p["f2w"], row2d(p["f2b"]), row2d(p["f2g"]), row2d(p["f2be"]),
        p["f3w"], row2d(p["f3b"]), row2d(p["f3g"]), row2d(p["f3be"]),
        p["f4w"], row2d(p["f4b"]),
    ]

    full = lambda shape: pl.BlockSpec(shape, lambda i: (0,) * len(shape))
    in_specs = [
        pl.BlockSpec((TB, D_QUERY), lambda i: (i, 0)),
        pl.BlockSpec((TB, NSLOTS, D_EMB), lambda i: (i, 0, 0)),
        pl.BlockSpec((1, NSLOTS, TB), lambda i: (i, 0, 0)),
        pl.BlockSpec((1, NSLOTS, TB), lambda i: (i, 0, 0)),
        pl.BlockSpec((1, NSLOTS, TB), lambda i: (i, 0, 0)),
    ]
    in_specs += [full(w.shape) for w in weights]

    out = pl.pallas_call(
        _fused_kernel,
        grid=(B // TB,),
        in_specs=in_specs,
        out_specs=pl.BlockSpec((TB, 1), lambda i: (i, 0)),
        out_shape=jax.ShapeDtypeStruct((B, 1), jnp.float32),
    )(q, xt, ip, il, ir, *weights)
    return out * lens[0].astype(out.dtype)


# speedup vs baseline: 1.5085x; 1.1678x over previous
import functools

import jax
import jax.numpy as jnp
from jax.experimental import pallas as pl

B = 256
NSLOTS = 128
M = NSLOTS - 1
D_EMB = 128
D_QUERY = 512
TB = 16

_F32 = jnp.float32
_BF16 = jnp.bfloat16


def _split3(a):
    hi = a.astype(_BF16)
    r1 = a - hi.astype(_F32)
    mid = r1.astype(_BF16)
    lo = (r1 - mid.astype(_F32)).astype(_BF16)
    return hi, mid, lo


def _fused_kernel(
    q_ref, x_ref, ip_ref, il_ref, ir_ref,
    q1w, q1b, q1g, q1be, q2w, q2b, q2g, q2be, q3w, q3b,
    w1, b1, w2, b2, w3, b3,
    f1w, f1b, f1g, f1be, f2w, f2b, f2g, f2be, f3w, f3b, f3g, f3be, f4w, f4b,
    out_ref,
):
    dot = functools.partial(jnp.dot, preferred_element_type=_F32)

    def ln(h, g, b):
        m = jnp.mean(h, axis=-1, keepdims=True)
        v = jnp.mean((h - m) ** 2, axis=-1, keepdims=True)
        return (h - m) / jnp.sqrt(v + 1e-5) * g + b

    h = jax.nn.relu(ln(dot(q_ref[...], q1w[...]) + q1b[...], q1g[...], q1be[...]))
    h = jax.nn.relu(ln(dot(h, q2w[...]) + q2b[...], q2g[...], q2be[...]))
    qf = dot(h, q3w[...]) + q3b[...]

    iota2 = jax.lax.broadcasted_iota(jnp.int32, (NSLOTS, 3 * NSLOTS), 1) & (NSLOTS - 1)
    rmask = jax.lax.broadcasted_iota(jnp.int32, (NSLOTS, 1), 0) > 0
    ohs = []
    for t in range(TB):
        oh3 = []
        for ref in (ip_ref, il_ref, ir_ref):
            col = ref[0, :, t : t + 1]
            oh3.append(((col == iota2) & rmask).astype(_BF16))
        ohs.append(oh3)
    rowmask = rmask.astype(_F32)

    xa = x_ref[...]
    trees = jnp.concatenate(
        [xa, jnp.broadcast_to(qf[:, None, :], (TB, NSLOTS, 32))], axis=2
    )

    def conv(tin, w, bias):
        cin = tin.shape[2]
        z = jnp.concatenate(_split3(tin), axis=1)
        rows = []
        for t in range(TB):
            zt = z[t]
            rows.append(
                jnp.concatenate(
                    [dot(ohs[t][0], zt), dot(ohs[t][1], zt), dot(ohs[t][2], zt)],
                    axis=1,
                )[None]
            )
        g = jnp.concatenate(rows, axis=0).reshape(TB * NSLOTS, 3 * cin)
        s = dot(g, w[...]).reshape(TB, NSLOTS, -1)
        return s + (rowmask * bias[...])[None]

    def tln(t):
        n = NSLOTS * t.shape[2]
        m = jnp.mean(t, axis=(1, 2), keepdims=True)
        d = t - m
        s = jnp.sqrt(jnp.sum(d * d, axis=(1, 2), keepdims=True) / (n - 1))
        return d / (s + 0.001)

    t1 = tln(conv(trees, w1, b1))
    t2 = tln(conv(t1, w2, b2))
    t3 = conv(t2, w3, b3)

    pooled = jnp.max(t3, axis=1)
    h = jax.nn.relu(ln(dot(pooled, f1w[...]) + f1b[...], f1g[...], f1be[...]))
    h = jax.nn.relu(ln(dot(h, f2w[...]) + f2b[...], f2g[...], f2be[...]))
    h = jax.nn.relu(ln(dot(h, f3w[...]) + f3b[...], f3g[...], f3be[...]))
    out_ref[...] = dot(h, f4w[...]) + f4b[...]


def kernel(q, x, indices, lens, params):
    p = params
    idx = indices[:, :, 0]
    zrow = jnp.zeros((1, B), jnp.int32)
    grp = lambda a: a.reshape(NSLOTS, B // TB, TB).transpose(1, 0, 2)
    ip = grp(jnp.concatenate([zrow, idx[0::3]], axis=0))
    il = grp(jnp.concatenate([zrow, idx[1::3]], axis=0))
    ir = grp(jnp.concatenate([zrow, idx[2::3]], axis=0))

    xt = jnp.transpose(x, (1, 0, 2))

    row2d = lambda a: a.reshape(1, -1)
    weights = [
        p["q1w"], row2d(p["q1b"]), row2d(p["q1g"]), row2d(p["q1be"]),
        p["q2w"], row2d(p["q2b"]), row2d(p["q2g"]), row2d(p["q2be"]),
        p["q3w"], row2d(p["q3b"]),
        p["c1w"], row2d(p["c1b"]),
        p["c2w"], row2d(p["c2b"]),
        p["c3w"], row2d(p["c3b"]),
        p["f1w"], row2d(p["f1b"]), row2d(p["f1g"]), row2d(p["f1be"]),
        p["f2w"], row2d(p["f2b"]), row2d(p["f2g"]), row2d(p["f2be"]),
        p["f3w"], row2d(p["f3b"]), row2d(p["f3g"]), row2d(p["f3be"]),
        p["f4w"], row2d(p["f4b"]),
    ]

    full = lambda shape: pl.BlockSpec(shape, lambda i: (0,) * len(shape))
    in_specs = [
        pl.BlockSpec((TB, D_QUERY), lambda i: (i, 0)),
        pl.BlockSpec((TB, NSLOTS, D_EMB), lambda i: (i, 0, 0)),
        pl.BlockSpec((1, NSLOTS, TB), lambda i: (i, 0, 0)),
        pl.BlockSpec((1, NSLOTS, TB), lambda i: (i, 0, 0)),
        pl.BlockSpec((1, NSLOTS, TB), lambda i: (i, 0, 0)),
    ]
    in_specs += [full(w.shape) for w in weights]

    out = pl.pallas_call(
        _fused_kernel,
        grid=(B // TB,),
        in_specs=in_specs,
        out_specs=pl.BlockSpec((TB, 1), lambda i: (i, 0)),
        out_shape=jax.ShapeDtypeStruct((B, 1), jnp.float32),
    )(q, xt, ip, il, ir, *weights)
    return out * lens[0].astype(out.dtype)


# speedup vs baseline: 1.6306x; 1.0809x over previous
import functools

import jax
import jax.numpy as jnp
from jax.experimental import pallas as pl

B = 256
NSLOTS = 128
M = NSLOTS - 1
D_EMB = 128
D_QUERY = 512
TB = 32

_F32 = jnp.float32
_BF16 = jnp.bfloat16


def _split3(a):
    hi = a.astype(_BF16)
    r1 = a - hi.astype(_F32)
    mid = r1.astype(_BF16)
    lo = (r1 - mid.astype(_F32)).astype(_BF16)
    return hi, mid, lo


def _fused_kernel(
    q_ref, x_ref, ip_ref, il_ref, ir_ref,
    q1w, q1b, q1g, q1be, q2w, q2b, q2g, q2be, q3w, q3b,
    w1, b1, w2, b2, w3, b3,
    f1w, f1b, f1g, f1be, f2w, f2b, f2g, f2be, f3w, f3b, f3g, f3be, f4w, f4b,
    out_ref,
):
    dot = functools.partial(jnp.dot, preferred_element_type=_F32)

    def ln(h, g, b):
        m = jnp.mean(h, axis=-1, keepdims=True)
        v = jnp.mean((h - m) ** 2, axis=-1, keepdims=True)
        return (h - m) / jnp.sqrt(v + 1e-5) * g + b

    h = jax.nn.relu(ln(dot(q_ref[...], q1w[...]) + q1b[...], q1g[...], q1be[...]))
    h = jax.nn.relu(ln(dot(h, q2w[...]) + q2b[...], q2g[...], q2be[...]))
    qf = dot(h, q3w[...]) + q3b[...]

    iota2 = jax.lax.broadcasted_iota(jnp.int32, (NSLOTS, 3 * NSLOTS), 1) & (NSLOTS - 1)
    rmask = jax.lax.broadcasted_iota(jnp.int32, (NSLOTS, 1), 0) > 0
    ohs = []
    for t in range(TB):
        oh3 = []
        for ref in (ip_ref, il_ref, ir_ref):
            col = ref[0, :, t : t + 1]
            oh3.append(((col == iota2) & rmask).astype(_BF16))
        ohs.append(oh3)
    rowmask = rmask.astype(_F32)

    xa = x_ref[...]
    trees = jnp.concatenate(
        [xa, jnp.broadcast_to(qf[:, None, :], (TB, NSLOTS, 32))], axis=2
    )

    def conv(tin, w, bias):
        cin = tin.shape[2]
        z = jnp.concatenate(_split3(tin), axis=1)
        rows = []
        for t in range(TB):
            zt = z[t]
            rows.append(
                jnp.concatenate(
                    [dot(ohs[t][0], zt), dot(ohs[t][1], zt), dot(ohs[t][2], zt)],
                    axis=1,
                )[None]
            )
        g = jnp.concatenate(rows, axis=0).reshape(TB * NSLOTS, 3 * cin)
        s = dot(g, w[...]).reshape(TB, NSLOTS, -1)
        return s + (rowmask * bias[...])[None]

    def tln(t):
        n = NSLOTS * t.shape[2]
        m = jnp.mean(t, axis=(1, 2), keepdims=True)
        d = t - m
        s = jnp.sqrt(jnp.sum(d * d, axis=(1, 2), keepdims=True) / (n - 1))
        return d / (s + 0.001)

    t1 = tln(conv(trees, w1, b1))
    t2 = tln(conv(t1, w2, b2))
    t3 = conv(t2, w3, b3)

    pooled = jnp.max(t3, axis=1)
    h = jax.nn.relu(ln(dot(pooled, f1w[...]) + f1b[...], f1g[...], f1be[...]))
    h = jax.nn.relu(ln(dot(h, f2w[...]) + f2b[...], f2g[...], f2be[...]))
    h = jax.nn.relu(ln(dot(h, f3w[...]) + f3b[...], f3g[...], f3be[...]))
    out_ref[...] = dot(h, f4w[...]) + f4b[...]


def kernel(q, x, indices, lens, params):
    p = params
    idx = indices[:, :, 0]
    zrow = jnp.zeros((1, B), jnp.int32)
    grp = lambda a: a.reshape(NSLOTS, B // TB, TB).transpose(1, 0, 2)
    ip = grp(jnp.concatenate([zrow, idx[0::3]], axis=0))
    il = grp(jnp.concatenate([zrow, idx[1::3]], axis=0))
    ir = grp(jnp.concatenate([zrow, idx[2::3]], axis=0))

    xt = jnp.transpose(x, (1, 0, 2))

    row2d = lambda a: a.reshape(1, -1)
    weights = [
        p["q1w"], row2d(p["q1b"]), row2d(p["q1g"]), row2d(p["q1be"]),
        p["q2w"], row2d(p["q2b"]), row2d(p["q2g"]), row2d(p["q2be"]),
        p["q3w"], row2d(p["q3b"]),
        p["c1w"], row2d(p["c1b"]),
        p["c2w"], row2d(p["c2b"]),
        p["c3w"], row2d(p["c3b"]),
        p["f1w"], row2d(p["f1b"]), row2d(p["f1g"]), row2d(p["f1be"]),
        p["f2w"], row2d(p["f2b"]), row2d(p["f2g"]), row2d(p["f2be"]),
        p["f3w"], row2d(p["f3b"]), row2d(p["f3g"]), row2d(p["f3be"]),
        p["f4w"], row2d(p["f4b"]),
    ]

    full = lambda shape: pl.BlockSpec(shape, lambda i: (0,) * len(shape))
    in_specs = [
        pl.BlockSpec((TB, D_QUERY), lambda i: (i, 0)),
        pl.BlockSpec((TB, NSLOTS, D_EMB), lambda i: (i, 0, 0)),
        pl.BlockSpec((1, NSLOTS, TB), lambda i: (i, 0, 0)),
        pl.BlockSpec((1, NSLOTS, TB), lambda i: (i, 0, 0)),
        pl.BlockSpec((1, NSLOTS, TB), lambda i: (i, 0, 0)),
    ]
    in_specs += [full(w.shape) for w in weights]

    out = pl.pallas_call(
        _fused_kernel,
        grid=(B // TB,),
        in_specs=in_specs,
        out_specs=pl.BlockSpec((TB, 1), lambda i: (i, 0)),
        out_shape=jax.ShapeDtypeStruct((B, 1), jnp.float32),
    )(q, xt, ip, il, ir, *weights)
    return out * lens[0].astype(out.dtype)


# speedup vs baseline: 1.6354x; 1.0029x over previous
import functools

import jax
import jax.numpy as jnp
from jax.experimental import pallas as pl
from jax.experimental.pallas import tpu as pltpu

B = 256
NSLOTS = 128
M = NSLOTS - 1
D_EMB = 128
D_QUERY = 512
TB = 32

_F32 = jnp.float32
_BF16 = jnp.bfloat16


def _split3(a):
    hi = a.astype(_BF16)
    r1 = a - hi.astype(_F32)
    mid = r1.astype(_BF16)
    lo = (r1 - mid.astype(_F32)).astype(_BF16)
    return hi, mid, lo


def _fused_kernel(
    q_ref, x_ref, ip_ref, il_ref, ir_ref,
    q1w, q1b, q1g, q1be, q2w, q2b, q2g, q2be, q3w, q3b,
    w1, b1, w2, b2, w3, b3,
    f1w, f1b, f1g, f1be, f2w, f2b, f2g, f2be, f3w, f3b, f3g, f3be, f4w, f4b,
    out_ref, g_ref,
):
    dot = functools.partial(jnp.dot, preferred_element_type=_F32)

    def ln(h, g, b):
        m = jnp.mean(h, axis=-1, keepdims=True)
        v = jnp.mean((h - m) ** 2, axis=-1, keepdims=True)
        return (h - m) / jnp.sqrt(v + 1e-5) * g + b

    h = jax.nn.relu(ln(dot(q_ref[...], q1w[...]) + q1b[...], q1g[...], q1be[...]))
    h = jax.nn.relu(ln(dot(h, q2w[...]) + q2b[...], q2g[...], q2be[...]))
    qf = dot(h, q3w[...]) + q3b[...]

    iota2 = jax.lax.broadcasted_iota(jnp.int32, (NSLOTS, 3 * NSLOTS), 1) & (NSLOTS - 1)
    rmask = jax.lax.broadcasted_iota(jnp.int32, (NSLOTS, 1), 0) > 0
    ohs = []
    for t in range(TB):
        oh3 = []
        for ref in (ip_ref, il_ref, ir_ref):
            col = ref[0, :, t : t + 1]
            oh3.append(((col == iota2) & rmask).astype(_BF16))
        ohs.append(oh3)
    rowmask = rmask.astype(_F32)

    xa = x_ref[...]
    trees = jnp.concatenate(
        [xa, jnp.broadcast_to(qf[:, None, :], (TB, NSLOTS, 32))], axis=2
    )

    def conv(tin, w, bias):
        cin = tin.shape[2]
        z = jnp.concatenate(_split3(tin), axis=1)
        for t in range(TB):
            zt = z[t]
            r0 = t * NSLOTS
            g_ref[r0 : r0 + NSLOTS, 0:cin] = dot(ohs[t][0], zt).astype(_BF16)
            g_ref[r0 : r0 + NSLOTS, cin : 2 * cin] = dot(ohs[t][1], zt).astype(_BF16)
            g_ref[r0 : r0 + NSLOTS, 2 * cin : 3 * cin] = dot(ohs[t][2], zt).astype(_BF16)
        s = dot(g_ref[:, 0 : 3 * cin], w[...]).reshape(TB, NSLOTS, -1)
        return s + (rowmask * bias[...])[None]

    def tln(t):
        n = NSLOTS * t.shape[2]
        m = jnp.mean(t, axis=(1, 2), keepdims=True)
        d = t - m
        s = jnp.sqrt(jnp.sum(d * d, axis=(1, 2), keepdims=True) / (n - 1))
        return d / (s + 0.001)

    t1 = tln(conv(trees, w1, b1))
    t2 = tln(conv(t1, w2, b2))
    t3 = conv(t2, w3, b3)

    pooled = jnp.max(t3, axis=1)
    h = jax.nn.relu(ln(dot(pooled, f1w[...]) + f1b[...], f1g[...], f1be[...]))
    h = jax.nn.relu(ln(dot(h, f2w[...]) + f2b[...], f2g[...], f2be[...]))
    h = jax.nn.relu(ln(dot(h, f3w[...]) + f3b[...], f3g[...], f3be[...]))
    out_ref[...] = dot(h, f4w[...]) + f4b[...]


def kernel(q, x, indices, lens, params):
    p = params
    idx = indices[:, :, 0]
    zrow = jnp.zeros((1, B), jnp.int32)
    grp = lambda a: a.reshape(NSLOTS, B // TB, TB).transpose(1, 0, 2)
    ip = grp(jnp.concatenate([zrow, idx[0::3]], axis=0))
    il = grp(jnp.concatenate([zrow, idx[1::3]], axis=0))
    ir = grp(jnp.concatenate([zrow, idx[2::3]], axis=0))

    xt = jnp.transpose(x, (1, 0, 2))

    row2d = lambda a: a.reshape(1, -1)
    weights = [
        p["q1w"], row2d(p["q1b"]), row2d(p["q1g"]), row2d(p["q1be"]),
        p["q2w"], row2d(p["q2b"]), row2d(p["q2g"]), row2d(p["q2be"]),
        p["q3w"], row2d(p["q3b"]),
        p["c1w"], row2d(p["c1b"]),
        p["c2w"], row2d(p["c2b"]),
        p["c3w"], row2d(p["c3b"]),
        p["f1w"], row2d(p["f1b"]), row2d(p["f1g"]), row2d(p["f1be"]),
        p["f2w"], row2d(p["f2b"]), row2d(p["f2g"]), row2d(p["f2be"]),
        p["f3w"], row2d(p["f3b"]), row2d(p["f3g"]), row2d(p["f3be"]),
        p["f4w"], row2d(p["f4b"]),
    ]

    full = lambda shape: pl.BlockSpec(shape, lambda i: (0,) * len(shape))
    in_specs = [
        pl.BlockSpec((TB, D_QUERY), lambda i: (i, 0)),
        pl.BlockSpec((TB, NSLOTS, D_EMB), lambda i: (i, 0, 0)),
        pl.BlockSpec((1, NSLOTS, TB), lambda i: (i, 0, 0)),
        pl.BlockSpec((1, NSLOTS, TB), lambda i: (i, 0, 0)),
        pl.BlockSpec((1, NSLOTS, TB), lambda i: (i, 0, 0)),
    ]
    in_specs += [full(w.shape) for w in weights]

    out = pl.pallas_call(
        _fused_kernel,
        grid=(B // TB,),
        in_specs=in_specs,
        out_specs=pl.BlockSpec((TB, 1), lambda i: (i, 0)),
        out_shape=jax.ShapeDtypeStruct((B, 1), jnp.float32),
        scratch_shapes=[pltpu.VMEM((TB * NSLOTS, 3 * 512), _BF16)],
    )(q, xt, ip, il, ir, *weights)
    return out * lens[0].astype(out.dtype)
